# baseline (speedup 1.0000x reference)
import functools

import jax
import jax.numpy as jnp
from jax.experimental import pallas as pl
from jax.experimental.pallas import tpu as pltpu

_VMEM_LIMIT = 40 * 1024 * 1024


def _mm_body(a_ref, b_ref, s_ref, o_ref, *, relu):
    y = jnp.dot(a_ref[...], b_ref[...], preferred_element_type=jnp.float32)
    y = y + s_ref[...]
    if relu:
        y = jnp.maximum(y, 0.0)
    o_ref[...] = y.astype(o_ref.dtype)


def _pick_bm(M):
    for cand in (512, 256, 128, 112, 64, 56, 32, 16, 8):
        if M % cand == 0:
            return cand
    return 8


def _matmul(a, b, shift, relu, out_dtype=jnp.bfloat16):
    M, K = a.shape
    N = b.shape[1]
    bm = _pick_bm(M)
    out = pl.pallas_call(
        functools.partial(_mm_body, relu=relu),
        out_shape=jax.ShapeDtypeStruct((M, N), out_dtype),
        grid=(M // bm,),
        in_specs=[
            pl.BlockSpec((bm, K), lambda i: (i, 0)),
            pl.BlockSpec((K, N), lambda i: (0, 0)),
            pl.BlockSpec((1, N), lambda i: (0, 0)),
        ],
        out_specs=pl.BlockSpec((bm, N), lambda i: (i, 0)),
        compiler_params=pltpu.CompilerParams(
            dimension_semantics=("arbitrary",),
            vmem_limit_bytes=_VMEM_LIMIT),
    )(a.astype(jnp.bfloat16), b, shift.astype(jnp.float32).reshape(1, N))
    return out


def _win_body(f0, f1, f2, w_ref, s_ref, *rest, Tp, Ho, Wo, C, relu, has_res):
    if has_res:
        res_ref, o_ref = rest
    else:
        (o_ref,) = rest
        res_ref = None
    tp = pl.program_id(1) % Tp
    o_ref[...] = jnp.zeros_like(o_ref)

    @pl.when(jnp.logical_and(tp >= 1, tp <= Tp - 2))
    def _():
        frames = (f0, f1, f2)
        tn = o_ref.shape[-1]
        acc = jnp.zeros((Ho * Wo, tn), jnp.float32)
        for kt in range(3):
            cur = frames[kt]
            for kh in range(3):
                for kw in range(3):
                    win = cur[0, 0, kh:kh + Ho, kw:kw + Wo, :]
                    acc = acc + jnp.dot(
                        win.reshape(Ho * Wo, C),
                        w_ref[kt * 9 + kh * 3 + kw],
                        preferred_element_type=jnp.float32)
        y = acc + s_ref[...]
        if res_ref is not None:
            y = y + res_ref[0, 0, 1:1 + Ho, 1:1 + Wo, :].reshape(
                Ho * Wo, tn).astype(jnp.float32)
        if relu:
            y = jnp.maximum(y, 0.0)
        o_ref[0, 0, 1:1 + Ho, 1:1 + Wo, :] = y.reshape(Ho, Wo, tn).astype(
            o_ref.dtype)


def _conv_s1(x, w_packed, shift, relu=True, residual=None):
    B, Tp, Hp, Wp, C = x.shape
    Ho, Wo = Hp - 2, Wp - 2
    Cout = w_packed.shape[1]
    tn = min(Cout, 256)
    nj = Cout // tn
    w = w_packed.reshape(27, C, Cout)
    has_res = residual is not None

    def frame_spec(k):
        return pl.BlockSpec(
            (1, 1, Hp, Wp, C),
            lambda j, m, k=k: (m // Tp,
                               jnp.clip(m % Tp - 1 + k, 0, Tp - 1), 0, 0, 0))

    in_specs = [frame_spec(0), frame_spec(1), frame_spec(2),
                pl.BlockSpec((27, C, tn), lambda j, m: (0, 0, j)),
                pl.BlockSpec((1, tn), lambda j, m: (0, j))]
    operands = [x, x, x, w, shift.astype(jnp.float32).reshape(1, Cout)]
    if has_res:
        in_specs.append(pl.BlockSpec(
            (1, 1, Hp, Wp, tn), lambda j, m: (m // Tp, m % Tp, 0, 0, j)))
        operands.append(residual)

    body = functools.partial(_win_body, Tp=Tp, Ho=Ho, Wo=Wo, C=C, relu=relu,
                             has_res=has_res)
    flops = 2 * B * (Tp - 2) * Ho * Wo * 27 * C * Cout
    out = pl.pallas_call(
        body,
        out_shape=jax.ShapeDtypeStruct((B, Tp, Hp, Wp, Cout), jnp.bfloat16),
        grid=(nj, B * Tp),
        in_specs=in_specs,
        out_specs=pl.BlockSpec(
            (1, 1, Hp, Wp, tn), lambda j, m: (m // Tp, m % Tp, 0, 0, j)),
        compiler_params=pltpu.CompilerParams(
            dimension_semantics=("parallel", "arbitrary"),
            vmem_limit_bytes=_VMEM_LIMIT),
        cost_estimate=pl.CostEstimate(flops=flops, transcendentals=0,
                                      bytes_accessed=6 * B * Tp * Hp * Wp * C),
    )(*operands)
    return out


def _fold_weights(w_packed, c):
    cout = w_packed.shape[1]
    w3 = w_packed.reshape(3, 3, 3, c, cout)
    z = jnp.zeros((c, cout), w_packed.dtype)
    taps = []
    for kt in range(3):
        for a in range(2):
            for b in range(2):
                rows = []
                for ph in range(2):
                    for pw in range(2):
                        cols = []
                        for po in range(2):
                            for qo in range(2):
                                kh = 2 * a + ph - po
                                kw = 2 * b + pw - qo
                                ok = 0 <= kh < 3 and 0 <= kw < 3
                                cols.append(w3[kt, kh, kw] if ok else z)
                        rows.append(jnp.concatenate(cols, axis=1))
                taps.append(jnp.concatenate(rows, axis=0))
    return jnp.stack(taps)


def _fold_body(f0, f1, f2, w_ref, s_ref, m_ref, *rest, Tp, Q, relu, has_res):
    if has_res:
        res_ref, o_ref = rest
    else:
        (o_ref,) = rest
        res_ref = None
    tp = pl.program_id(0) % Tp
    Qo = Q - 1
    o_ref[...] = jnp.zeros_like(o_ref)

    @pl.when(jnp.logical_and(tp >= 1, tp <= Tp - 2))
    def _():
        frames = (f0, f1, f2)
        cf = w_ref.shape[1]
        acc = jnp.zeros((Qo * Qo, cf), jnp.float32)
        for kt in range(3):
            cur = frames[kt]
            for a in range(2):
                for b in range(2):
                    win = cur[0, 0, a:a + Qo, b:b + Qo, :]
                    acc = acc + jnp.dot(
                        win.reshape(Qo * Qo, cf),
                        w_ref[(kt * 2 + a) * 2 + b],
                        preferred_element_type=jnp.float32)
        y = acc + s_ref[...]
        if res_ref is not None:
            y = y + res_ref[0, 0, 1:1 + Qo, 1:1 + Qo, :].reshape(
                Qo * Qo, cf).astype(jnp.float32)
        if relu:
            y = jnp.maximum(y, 0.0)
        y = y * m_ref[...]
        o_ref[0, 0, :Qo, :Qo, :] = y.reshape(Qo, Qo, cf).astype(o_ref.dtype)


def _conv_s1_folded(x, w_folded, shift4, mask, relu=True, residual=None):
    B, Tp, Q, _, CF = x.shape
    has_res = residual is not None

    def frame_spec(k):
        return pl.BlockSpec(
            (1, 1, Q, Q, CF),
            lambda m, k=k: (m // Tp,
                            jnp.clip(m % Tp - 1 + k, 0, Tp - 1), 0, 0, 0))

    in_specs = [frame_spec(0), frame_spec(1), frame_spec(2),
                pl.BlockSpec((12, CF, CF), lambda m: (0, 0, 0)),
                pl.BlockSpec((1, CF), lambda m: (0, 0)),
                pl.BlockSpec(((Q - 1) * (Q - 1), CF), lambda m: (0, 0))]
    operands = [x, x, x, w_folded, shift4.reshape(1, CF), mask]
    if has_res:
        in_specs.append(pl.BlockSpec(
            (1, 1, Q, Q, CF), lambda m: (m // Tp, m % Tp, 0, 0, 0)))
        operands.append(residual)

    body = functools.partial(_fold_body, Tp=Tp, Q=Q, relu=relu,
                             has_res=has_res)
    flops = 2 * B * (Tp - 2) * (Q - 1) * (Q - 1) * 12 * CF * CF
    out = pl.pallas_call(
        body,
        out_shape=jax.ShapeDtypeStruct((B, Tp, Q, Q, CF), jnp.bfloat16),
        grid=(B * Tp,),
        in_specs=in_specs,
        out_specs=pl.BlockSpec(
            (1, 1, Q, Q, CF), lambda m: (m // Tp, m % Tp, 0, 0, 0)),
        compiler_params=pltpu.CompilerParams(
            dimension_semantics=("arbitrary",),
            vmem_limit_bytes=_VMEM_LIMIT),
        cost_estimate=pl.CostEstimate(flops=flops, transcendentals=0,
                                      bytes_accessed=6 * B * Tp * Q * Q * CF),
    )(*operands)
    return out


def _fold_mask(Q, alpha):
    Qo = Q - 1
    H = 2 * Q - 8
    q = jnp.arange(Qo)[:, None]
    ph = jnp.arange(2)[None, :]
    o = 2 * q + ph - alpha
    ok = jnp.logical_and(o >= 0, o <= H - 1)
    m = (ok[:, None, :, None] & ok[None, :, None, :]).astype(jnp.float32)
    return m.reshape(Qo * Qo, 4)


def _fold_activation(x, c):
    B, T, H, W, _ = x.shape
    xp = jnp.pad(x, ((0, 0), (1, 1), (5, 3), (5, 3), (0, 0)))
    Q = (H + 8) // 2
    xp = xp.reshape(B, T + 2, Q, 2, Q, 2, c)
    xp = xp.transpose(0, 1, 2, 4, 3, 5, 6)
    return xp.reshape(B, T + 2, Q, Q, 4 * c)


def _unfold_activation(xf, c, alpha):
    B, Tp, Q, _, _ = xf.shape
    H = 2 * Q - 8
    x = xf.reshape(B, Tp, Q, Q, 2, 2, c)
    x = x.transpose(0, 1, 2, 4, 3, 5, 6)
    x = x.reshape(B, Tp, 2 * Q, 2 * Q, c)
    return x[:, :, alpha - 1:alpha + H + 1, alpha - 1:alpha + H + 1, :]


def _im2col_s2(xp, To, Ho, Wo, kt_n=3, kh_n=3, kw_n=3):
    C = xp.shape[-1]
    B = xp.shape[0]
    cols = []
    for kt in range(kt_n):
        for kh in range(kh_n):
            for kw in range(kw_n):
                cols.append(xp[:, kt:kt + 2 * To - 1:2,
                               kh:kh + 2 * Ho - 1:2,
                               kw:kw + 2 * Wo - 1:2, :])
    P = jnp.stack(cols, axis=4)
    return P.reshape(B * To * Ho * Wo, kt_n * kh_n * kw_n * C)


def _pad_act(y, B, To, Ho, Wo, C):
    y = y.reshape(B, To, Ho, Wo, C)
    return jnp.pad(y, ((0, 0), (1, 1), (1, 1), (1, 1), (0, 0)))


def _block_pair(xpad, w1, s1, w2, s2, down=None):
    out = _conv_s1(xpad, w1, s1, relu=True)
    identity = xpad if down is None else down
    return _conv_s1(out, w2, s2, relu=True, residual=identity)


def kernel(x, stem_w, stem_shift,
           layer1_b0_conv1_w, layer1_b0_conv1_shift,
           layer1_b0_conv2_w, layer1_b0_conv2_shift,
           layer1_b1_conv1_w, layer1_b1_conv1_shift,
           layer1_b1_conv2_w, layer1_b1_conv2_shift,
           layer2_b0_conv1_w, layer2_b0_conv1_shift,
           layer2_b0_conv2_w, layer2_b0_conv2_shift,
           layer2_b0_down_w, layer2_b0_down_shift,
           layer2_b1_conv1_w, layer2_b1_conv1_shift,
           layer2_b1_conv2_w, layer2_b1_conv2_shift,
           layer3_b0_conv1_w, layer3_b0_conv1_shift,
           layer3_b0_conv2_w, layer3_b0_conv2_shift,
           layer3_b0_down_w, layer3_b0_down_shift,
           layer3_b1_conv1_w, layer3_b1_conv1_shift,
           layer3_b1_conv2_w, layer3_b1_conv2_shift,
           layer4_b0_conv1_w, layer4_b0_conv1_shift,
           layer4_b0_conv2_w, layer4_b0_conv2_shift,
           layer4_b0_down_w, layer4_b0_down_shift,
           layer4_b1_conv1_w, layer4_b1_conv1_shift,
           layer4_b1_conv2_w, layer4_b1_conv2_shift,
           proj_w, proj_shift):
    B = x.shape[0]

    xs = jnp.transpose(x, (0, 2, 3, 4, 1)).astype(jnp.bfloat16)[..., 0]
    xs = jnp.pad(xs, ((0, 0), (1, 1), (3, 3), (3, 3)))
    cols = []
    for kt in range(3):
        for kh in range(7):
            for kw in range(7):
                cols.append(xs[:, kt:kt + 16,
                               kh:kh + 111:2, kw:kw + 111:2])
    A = jnp.stack(cols, axis=-1).reshape(B * 16 * 56 * 56, 147)
    stem = _matmul(A, stem_w, stem_shift, relu=True)

    f = _fold_activation(stem.reshape(B, 16, 56, 56, 64), 64)
    Q = f.shape[2]
    masks = [jnp.repeat(_fold_mask(Q, a), 64, axis=1) for a in (4, 3, 2, 1)]
    sh = lambda s: jnp.tile(s.astype(jnp.float32), 4)
    w10c1 = _fold_weights(layer1_b0_conv1_w, 64)
    w10c2 = _fold_weights(layer1_b0_conv2_w, 64)
    w11c1 = _fold_weights(layer1_b1_conv1_w, 64)
    w11c2 = _fold_weights(layer1_b1_conv2_w, 64)
    t = _conv_s1_folded(f, w10c1, sh(layer1_b0_conv1_shift), masks[0])
    f1o = _conv_s1_folded(t, w10c2, sh(layer1_b0_conv2_shift), masks[1],
                          residual=f)
    t = _conv_s1_folded(f1o, w11c1, sh(layer1_b1_conv1_shift), masks[2])
    act1 = _conv_s1_folded(t, w11c2, sh(layer1_b1_conv2_shift), masks[3],
                           residual=f1o)
    act1u = _unfold_activation(act1, 64, 1)

    A = _im2col_s2(act1u, 8, 28, 28)
    y = _matmul(A, layer2_b0_conv1_w, layer2_b0_conv1_shift, relu=True)
    c1p = _pad_act(y, B, 8, 28, 28, 128)
    Ad = act1u[:, 1:16:2, 1:56:2, 1:56:2, :].reshape(-1, 64)
    yd = _matmul(Ad, layer2_b0_down_w, layer2_b0_down_shift, relu=False)
    dpad = _pad_act(yd, B, 8, 28, 28, 128)
    b0 = _conv_s1(c1p, layer2_b0_conv2_w, layer2_b0_conv2_shift, relu=True,
                  residual=dpad)
    act2 = _block_pair(b0, layer2_b1_conv1_w, layer2_b1_conv1_shift,
                       layer2_b1_conv2_w, layer2_b1_conv2_shift)

    A = _im2col_s2(act2, 4, 14, 14)
    y = _matmul(A, layer3_b0_conv1_w, layer3_b0_conv1_shift, relu=True)
    c1p = _pad_act(y, B, 4, 14, 14, 256)
    Ad = act2[:, 1:8:2, 1:28:2, 1:28:2, :].reshape(-1, 128)
    yd = _matmul(Ad, layer3_b0_down_w, layer3_b0_down_shift, relu=False)
    dpad = _pad_act(yd, B, 4, 14, 14, 256)
    b0 = _conv_s1(c1p, layer3_b0_conv2_w, layer3_b0_conv2_shift, relu=True,
                  residual=dpad)
    act3 = _block_pair(b0, layer3_b1_conv1_w, layer3_b1_conv1_shift,
                       layer3_b1_conv2_w, layer3_b1_conv2_shift)

    A = _im2col_s2(act3, 2, 7, 7)
    y = _matmul(A, layer4_b0_conv1_w, layer4_b0_conv1_shift, relu=True)
    c1p = _pad_act(y, B, 2, 7, 7, 512)
    Ad = act3[:, 1:4:2, 1:14:2, 1:14:2, :].reshape(-1, 256)
    yd = _matmul(Ad, layer4_b0_down_w, layer4_b0_down_shift, relu=False)
    dpad = _pad_act(yd, B, 2, 7, 7, 512)
    b0 = _conv_s1(c1p, layer4_b0_conv2_w, layer4_b0_conv2_shift, relu=True,
                  residual=dpad)
    act4 = _block_pair(b0, layer4_b1_conv1_w, layer4_b1_conv1_shift,
                       layer4_b1_conv2_w, layer4_b1_conv2_shift)

    feats = jnp.mean(act4[:, 1:3, 1:8, 1:8, :].astype(jnp.float32),
                     axis=(1, 2, 3))
    emb = _matmul(feats, proj_w, proj_shift, relu=False, out_dtype=jnp.float32)
    return emb


# speedup vs baseline: 1.0266x; 1.0266x over previous
import functools

import jax
import jax.numpy as jnp
from jax.experimental import pallas as pl
from jax.experimental.pallas import tpu as pltpu

_VMEM_LIMIT = 40 * 1024 * 1024


def _mm_body(a_ref, b_ref, s_ref, o_ref, *, relu):
    y = jnp.dot(a_ref[...], b_ref[...], preferred_element_type=jnp.float32)
    y = y + s_ref[...]
    if relu:
        y = jnp.maximum(y, 0.0)
    o_ref[...] = y.astype(o_ref.dtype)


def _pick_bm(M):
    for cand in (512, 256, 128, 112, 64, 56, 32, 16, 8):
        if M % cand == 0:
            return cand
    return 8


def _matmul(a, b, shift, relu, out_dtype=jnp.bfloat16):
    M, K = a.shape
    N = b.shape[1]
    bm = _pick_bm(M)
    out = pl.pallas_call(
        functools.partial(_mm_body, relu=relu),
        out_shape=jax.ShapeDtypeStruct((M, N), out_dtype),
        grid=(M // bm,),
        in_specs=[
            pl.BlockSpec((bm, K), lambda i: (i, 0)),
            pl.BlockSpec((K, N), lambda i: (0, 0)),
            pl.BlockSpec((1, N), lambda i: (0, 0)),
        ],
        out_specs=pl.BlockSpec((bm, N), lambda i: (i, 0)),
        compiler_params=pltpu.CompilerParams(
            dimension_semantics=("arbitrary",),
            vmem_limit_bytes=_VMEM_LIMIT),
    )(a.astype(jnp.bfloat16), b, shift.astype(jnp.float32).reshape(1, N))
    return out


def _win_body(f0, f1, f2, w_ref, s_ref, *rest, Tp, Ho, Wo, C, relu, has_res):
    if has_res:
        res_ref, o_ref = rest
    else:
        (o_ref,) = rest
        res_ref = None
    tp = pl.program_id(1) % Tp
    o_ref[...] = jnp.zeros_like(o_ref)

    @pl.when(jnp.logical_and(tp >= 1, tp <= Tp - 2))
    def _():
        frames = (f0, f1, f2)
        Hp, Wp = Ho + 3, Wo + 2
        tn = o_ref.shape[-1]
        M = Ho * Wp
        acc = jnp.zeros((M, tn), jnp.float32)
        for kt in range(3):
            flat = frames[kt][0, 0].reshape(Hp * Wp, C)
            for kh in range(3):
                for kw in range(3):
                    off = kh * Wp + kw
                    acc = acc + jnp.dot(
                        flat[off:off + M],
                        w_ref[kt * 9 + kh * 3 + kw],
                        preferred_element_type=jnp.float32)
        y = acc + s_ref[...]
        if res_ref is not None:
            rflat = res_ref[0, 0].reshape(Hp * Wp, tn)
            y = y + rflat[Wp + 1:Wp + 1 + M].astype(jnp.float32)
        if relu:
            y = jnp.maximum(y, 0.0)
        y3 = y.reshape(Ho, Wp, tn)
        o_ref[0, 0, 1:1 + Ho, 1:1 + Wo, :] = y3[:, 0:Wo, :].astype(o_ref.dtype)


def _conv_s1(x, w_packed, shift, relu=True, residual=None):
    B, Tp, Hp, Wp, C = x.shape
    Ho, Wo = Hp - 3, Wp - 2
    Cout = w_packed.shape[1]
    tn = min(Cout, 256)
    nj = Cout // tn
    w = w_packed.reshape(27, C, Cout)
    has_res = residual is not None

    def frame_spec(k):
        return pl.BlockSpec(
            (1, 1, Hp, Wp, C),
            lambda j, m, k=k: (m // Tp,
                               jnp.clip(m % Tp - 1 + k, 0, Tp - 1), 0, 0, 0))

    in_specs = [frame_spec(0), frame_spec(1), frame_spec(2),
                pl.BlockSpec((27, C, tn), lambda j, m: (0, 0, j)),
                pl.BlockSpec((1, tn), lambda j, m: (0, j))]
    operands = [x, x, x, w, shift.astype(jnp.float32).reshape(1, Cout)]
    if has_res:
        in_specs.append(pl.BlockSpec(
            (1, 1, Hp, Wp, tn), lambda j, m: (m // Tp, m % Tp, 0, 0, j)))
        operands.append(residual)

    body = functools.partial(_win_body, Tp=Tp, Ho=Ho, Wo=Wo, C=C, relu=relu,
                             has_res=has_res)
    flops = 2 * B * (Tp - 2) * Ho * Wo * 27 * C * Cout
    out = pl.pallas_call(
        body,
        out_shape=jax.ShapeDtypeStruct((B, Tp, Hp, Wp, Cout), jnp.bfloat16),
        grid=(nj, B * Tp),
        in_specs=in_specs,
        out_specs=pl.BlockSpec(
            (1, 1, Hp, Wp, tn), lambda j, m: (m // Tp, m % Tp, 0, 0, j)),
        compiler_params=pltpu.CompilerParams(
            dimension_semantics=("parallel", "arbitrary"),
            vmem_limit_bytes=_VMEM_LIMIT),
        cost_estimate=pl.CostEstimate(flops=flops, transcendentals=0,
                                      bytes_accessed=6 * B * Tp * Hp * Wp * C),
    )(*operands)
    return out


def _fold_weights(w_packed, c):
    cout = w_packed.shape[1]
    w3 = w_packed.reshape(3, 3, 3, c, cout)
    z = jnp.zeros((c, cout), w_packed.dtype)
    taps = []
    for kt in range(3):
        for a in range(2):
            for b in range(2):
                rows = []
                for ph in range(2):
                    for pw in range(2):
                        cols = []
                        for po in range(2):
                            for qo in range(2):
                                kh = 2 * a + ph - po
                                kw = 2 * b + pw - qo
                                ok = 0 <= kh < 3 and 0 <= kw < 3
                                cols.append(w3[kt, kh, kw] if ok else z)
                        rows.append(jnp.concatenate(cols, axis=1))
                taps.append(jnp.concatenate(rows, axis=0))
    return jnp.stack(taps)


def _fold_body(f0, f1, f2, w_ref, s_ref, m_ref, *rest, Tp, Q, relu, has_res):
    if has_res:
        res_ref, o_ref = rest
    else:
        (o_ref,) = rest
        res_ref = None
    tp = pl.program_id(0) % Tp
    Qc = Q - 2
    M = Qc * Q
    o_ref[...] = jnp.zeros_like(o_ref)

    @pl.when(jnp.logical_and(tp >= 1, tp <= Tp - 2))
    def _():
        frames = (f0, f1, f2)
        cf = w_ref.shape[1]
        acc = jnp.zeros((M, cf), jnp.float32)
        for kt in range(3):
            flat = frames[kt][0, 0].reshape(Q * Q, cf)
            for a in range(2):
                for b in range(2):
                    off = a * Q + b
                    acc = acc + jnp.dot(
                        flat[off:off + M],
                        w_ref[(kt * 2 + a) * 2 + b],
                        preferred_element_type=jnp.float32)
        y = acc + s_ref[...]
        if res_ref is not None:
            rflat = res_ref[0, 0].reshape(Q * Q, cf)
            y = y + rflat[Q + 1:Q + 1 + M].astype(jnp.float32)
        if relu:
            y = jnp.maximum(y, 0.0)
        y = y * m_ref[...]
        o_ref[0, 0, 0:Qc, :, :] = y.reshape(Qc, Q, cf).astype(o_ref.dtype)


def _conv_s1_folded(x, w_folded, shift4, mask, relu=True, residual=None):
    B, Tp, Q, _, CF = x.shape
    has_res = residual is not None

    def frame_spec(k):
        return pl.BlockSpec(
            (1, 1, Q, Q, CF),
            lambda m, k=k: (m // Tp,
                            jnp.clip(m % Tp - 1 + k, 0, Tp - 1), 0, 0, 0))

    in_specs = [frame_spec(0), frame_spec(1), frame_spec(2),
                pl.BlockSpec((12, CF, CF), lambda m: (0, 0, 0)),
                pl.BlockSpec((1, CF), lambda m: (0, 0)),
                pl.BlockSpec(((Q - 2) * Q, CF), lambda m: (0, 0))]
    operands = [x, x, x, w_folded, shift4.reshape(1, CF), mask]
    if has_res:
        in_specs.append(pl.BlockSpec(
            (1, 1, Q, Q, CF), lambda m: (m // Tp, m % Tp, 0, 0, 0)))
        operands.append(residual)

    body = functools.partial(_fold_body, Tp=Tp, Q=Q, relu=relu,
                             has_res=has_res)
    flops = 2 * B * (Tp - 2) * (Q - 1) * (Q - 1) * 12 * CF * CF
    out = pl.pallas_call(
        body,
        out_shape=jax.ShapeDtypeStruct((B, Tp, Q, Q, CF), jnp.bfloat16),
        grid=(B * Tp,),
        in_specs=in_specs,
        out_specs=pl.BlockSpec(
            (1, 1, Q, Q, CF), lambda m: (m // Tp, m % Tp, 0, 0, 0)),
        compiler_params=pltpu.CompilerParams(
            dimension_semantics=("arbitrary",),
            vmem_limit_bytes=_VMEM_LIMIT),
        cost_estimate=pl.CostEstimate(flops=flops, transcendentals=0,
                                      bytes_accessed=6 * B * Tp * Q * Q * CF),
    )(*operands)
    return out


def _fold_mask(Q, alpha):
    Qc = Q - 2
    H = 2 * Q - 8
    ph = jnp.arange(2)[None, :]
    oh = 2 * jnp.arange(Qc)[:, None] + ph - alpha
    ow = 2 * jnp.arange(Q)[:, None] + ph - alpha
    okh = jnp.logical_and(oh >= 0, oh <= H - 1)
    okw = jnp.logical_and(ow >= 0, ow <= H - 1)
    m = (okh[:, None, :, None] & okw[None, :, None, :]).astype(jnp.float32)
    return m.reshape(Qc * Q, 4)


def _fold_activation(x, c):
    B, T, H, W, _ = x.shape
    xp = jnp.pad(x, ((0, 0), (1, 1), (5, 3), (5, 3), (0, 0)))
    Q = (H + 8) // 2
    xp = xp.reshape(B, T + 2, Q, 2, Q, 2, c)
    xp = xp.transpose(0, 1, 2, 4, 3, 5, 6)
    return xp.reshape(B, T + 2, Q, Q, 4 * c)


def _unfold_activation(xf, c, alpha):
    B, Tp, Q, _, _ = xf.shape
    H = 2 * Q - 8
    x = xf.reshape(B, Tp, Q, Q, 2, 2, c)
    x = x.transpose(0, 1, 2, 4, 3, 5, 6)
    x = x.reshape(B, Tp, 2 * Q, 2 * Q, c)
    return x[:, :, alpha - 1:alpha + H + 1, alpha - 1:alpha + H + 1, :]


def _im2col_s2(xp, To, Ho, Wo, kt_n=3, kh_n=3, kw_n=3):
    C = xp.shape[-1]
    B = xp.shape[0]
    cols = []
    for kt in range(kt_n):
        for kh in range(kh_n):
            for kw in range(kw_n):
                cols.append(xp[:, kt:kt + 2 * To - 1:2,
                               kh:kh + 2 * Ho - 1:2,
                               kw:kw + 2 * Wo - 1:2, :])
    P = jnp.stack(cols, axis=4)
    return P.reshape(B * To * Ho * Wo, kt_n * kh_n * kw_n * C)


def _pad_act(y, B, To, Ho, Wo, C):
    y = y.reshape(B, To, Ho, Wo, C)
    return jnp.pad(y, ((0, 0), (1, 1), (1, 2), (1, 1), (0, 0)))


def _block_pair(xpad, w1, s1, w2, s2, down=None):
    out = _conv_s1(xpad, w1, s1, relu=True)
    identity = xpad if down is None else down
    return _conv_s1(out, w2, s2, relu=True, residual=identity)


def kernel(x, stem_w, stem_shift,
           layer1_b0_conv1_w, layer1_b0_conv1_shift,
           layer1_b0_conv2_w, layer1_b0_conv2_shift,
           layer1_b1_conv1_w, layer1_b1_conv1_shift,
           layer1_b1_conv2_w, layer1_b1_conv2_shift,
           layer2_b0_conv1_w, layer2_b0_conv1_shift,
           layer2_b0_conv2_w, layer2_b0_conv2_shift,
           layer2_b0_down_w, layer2_b0_down_shift,
           layer2_b1_conv1_w, layer2_b1_conv1_shift,
           layer2_b1_conv2_w, layer2_b1_conv2_shift,
           layer3_b0_conv1_w, layer3_b0_conv1_shift,
           layer3_b0_conv2_w, layer3_b0_conv2_shift,
           layer3_b0_down_w, layer3_b0_down_shift,
           layer3_b1_conv1_w, layer3_b1_conv1_shift,
           layer3_b1_conv2_w, layer3_b1_conv2_shift,
           layer4_b0_conv1_w, layer4_b0_conv1_shift,
           layer4_b0_conv2_w, layer4_b0_conv2_shift,
           layer4_b0_down_w, layer4_b0_down_shift,
           layer4_b1_conv1_w, layer4_b1_conv1_shift,
           layer4_b1_conv2_w, layer4_b1_conv2_shift,
           proj_w, proj_shift):
    B = x.shape[0]

    xs = jnp.transpose(x, (0, 2, 3, 4, 1)).astype(jnp.bfloat16)[..., 0]
    xs = jnp.pad(xs, ((0, 0), (1, 1), (3, 3), (3, 3)))
    cols = []
    for kt in range(3):
        for kh in range(7):
            for kw in range(7):
                cols.append(xs[:, kt:kt + 16,
                               kh:kh + 111:2, kw:kw + 111:2])
    A = jnp.stack(cols, axis=-1).reshape(B * 16 * 56 * 56, 147)
    stem = _matmul(A, stem_w, stem_shift, relu=True)

    f = _fold_activation(stem.reshape(B, 16, 56, 56, 64), 64)
    Q = f.shape[2]
    masks = [jnp.repeat(_fold_mask(Q, a), 64, axis=1) for a in (4, 3, 2, 1)]
    sh = lambda s: jnp.tile(s.astype(jnp.float32), 4)
    w10c1 = _fold_weights(layer1_b0_conv1_w, 64)
    w10c2 = _fold_weights(layer1_b0_conv2_w, 64)
    w11c1 = _fold_weights(layer1_b1_conv1_w, 64)
    w11c2 = _fold_weights(layer1_b1_conv2_w, 64)
    t = _conv_s1_folded(f, w10c1, sh(layer1_b0_conv1_shift), masks[0])
    f1o = _conv_s1_folded(t, w10c2, sh(layer1_b0_conv2_shift), masks[1],
                          residual=f)
    t = _conv_s1_folded(f1o, w11c1, sh(layer1_b1_conv1_shift), masks[2])
    act1 = _conv_s1_folded(t, w11c2, sh(layer1_b1_conv2_shift), masks[3],
                           residual=f1o)
    act1u = _unfold_activation(act1, 64, 1)

    A = _im2col_s2(act1u, 8, 28, 28)
    y = _matmul(A, layer2_b0_conv1_w, layer2_b0_conv1_shift, relu=True)
    c1p = _pad_act(y, B, 8, 28, 28, 128)
    Ad = act1u[:, 1:16:2, 1:56:2, 1:56:2, :].reshape(-1, 64)
    yd = _matmul(Ad, layer2_b0_down_w, layer2_b0_down_shift, relu=False)
    dpad = _pad_act(yd, B, 8, 28, 28, 128)
    b0 = _conv_s1(c1p, layer2_b0_conv2_w, layer2_b0_conv2_shift, relu=True,
                  residual=dpad)
    act2 = _block_pair(b0, layer2_b1_conv1_w, layer2_b1_conv1_shift,
                       layer2_b1_conv2_w, layer2_b1_conv2_shift)

    A = _im2col_s2(act2, 4, 14, 14)
    y = _matmul(A, layer3_b0_conv1_w, layer3_b0_conv1_shift, relu=True)
    c1p = _pad_act(y, B, 4, 14, 14, 256)
    Ad = act2[:, 1:8:2, 1:28:2, 1:28:2, :].reshape(-1, 128)
    yd = _matmul(Ad, layer3_b0_down_w, layer3_b0_down_shift, relu=False)
    dpad = _pad_act(yd, B, 4, 14, 14, 256)
    b0 = _conv_s1(c1p, layer3_b0_conv2_w, layer3_b0_conv2_shift, relu=True,
                  residual=dpad)
    act3 = _block_pair(b0, layer3_b1_conv1_w, layer3_b1_conv1_shift,
                       layer3_b1_conv2_w, layer3_b1_conv2_shift)

    A = _im2col_s2(act3, 2, 7, 7)
    y = _matmul(A, layer4_b0_conv1_w, layer4_b0_conv1_shift, relu=True)
    c1p = _pad_act(y, B, 2, 7, 7, 512)
    Ad = act3[:, 1:4:2, 1:14:2, 1:14:2, :].reshape(-1, 256)
    yd = _matmul(Ad, layer4_b0_down_w, layer4_b0_down_shift, relu=False)
    dpad = _pad_act(yd, B, 2, 7, 7, 512)
    b0 = _conv_s1(c1p, layer4_b0_conv2_w, layer4_b0_conv2_shift, relu=True,
                  residual=dpad)
    act4 = _block_pair(b0, layer4_b1_conv1_w, layer4_b1_conv1_shift,
                       layer4_b1_conv2_w, layer4_b1_conv2_shift)

    feats = jnp.mean(act4[:, 1:3, 1:8, 1:8, :].astype(jnp.float32),
                     axis=(1, 2, 3))
    emb = _matmul(feats, proj_w, proj_shift, relu=False, out_dtype=jnp.float32)
    return emb


# speedup vs baseline: 1.0300x; 1.0033x over previous
import functools

import jax
import jax.numpy as jnp
from jax.experimental import pallas as pl
from jax.experimental.pallas import tpu as pltpu

_VMEM_LIMIT = 40 * 1024 * 1024


def _mm_body(a_ref, b_ref, s_ref, o_ref, *, relu):
    y = jnp.dot(a_ref[...], b_ref[...], preferred_element_type=jnp.float32)
    y = y + s_ref[...]
    if relu:
        y = jnp.maximum(y, 0.0)
    o_ref[...] = y.astype(o_ref.dtype)


def _pick_bm(M):
    for cand in (1024, 512, 256, 128, 112, 64, 56, 32, 16, 8):
        if M % cand == 0:
            return cand
    return 8


def _matmul(a, b, shift, relu, out_dtype=jnp.bfloat16):
    M, K = a.shape
    N = b.shape[1]
    bm = _pick_bm(M)
    out = pl.pallas_call(
        functools.partial(_mm_body, relu=relu),
        out_shape=jax.ShapeDtypeStruct((M, N), out_dtype),
        grid=(M // bm,),
        in_specs=[
            pl.BlockSpec((bm, K), lambda i: (i, 0)),
            pl.BlockSpec((K, N), lambda i: (0, 0)),
            pl.BlockSpec((1, N), lambda i: (0, 0)),
        ],
        out_specs=pl.BlockSpec((bm, N), lambda i: (i, 0)),
        compiler_params=pltpu.CompilerParams(
            dimension_semantics=("arbitrary",),
            vmem_limit_bytes=_VMEM_LIMIT),
    )(a.astype(jnp.bfloat16), b, shift.astype(jnp.float32).reshape(1, N))
    return out


def _win_body(f0, f1, f2, w_ref, s_ref, *rest, Tp, Ho, Wo, C, relu, has_res):
    if has_res:
        res_ref, o_ref = rest
    else:
        (o_ref,) = rest
        res_ref = None
    tp = pl.program_id(1) % Tp
    o_ref[...] = jnp.zeros_like(o_ref)

    @pl.when(jnp.logical_and(tp >= 1, tp <= Tp - 2))
    def _():
        frames = (f0, f1, f2)
        Hp, Wp = Ho + 3, Wo + 2
        tn = o_ref.shape[-1]
        M = Ho * Wp
        acc = jnp.zeros((M, tn), jnp.float32)
        for kt in range(3):
            flat = frames[kt][0, 0].reshape(Hp * Wp, C)
            for kh in range(3):
                for kw in range(3):
                    off = kh * Wp + kw
                    acc = acc + jnp.dot(
                        flat[off:off + M],
                        w_ref[kt * 9 + kh * 3 + kw],
                        preferred_element_type=jnp.float32)
        y = acc + s_ref[...]
        if res_ref is not None:
            rflat = res_ref[0, 0].reshape(Hp * Wp, tn)
            y = y + rflat[Wp + 1:Wp + 1 + M].astype(jnp.float32)
        if relu:
            y = jnp.maximum(y, 0.0)
        y3 = y.reshape(Ho, Wp, tn)
        o_ref[0, 0, 1:1 + Ho, 1:1 + Wo, :] = y3[:, 0:Wo, :].astype(o_ref.dtype)


def _conv_s1(x, w_packed, shift, relu=True, residual=None):
    B, Tp, Hp, Wp, C = x.shape
    Ho, Wo = Hp - 3, Wp - 2
    Cout = w_packed.shape[1]
    tn = min(Cout, 256)
    nj = Cout // tn
    w = w_packed.reshape(27, C, Cout)
    has_res = residual is not None

    def frame_spec(k):
        return pl.BlockSpec(
            (1, 1, Hp, Wp, C),
            lambda j, m, k=k: (m // Tp,
                               jnp.clip(m % Tp - 1 + k, 0, Tp - 1), 0, 0, 0))

    in_specs = [frame_spec(0), frame_spec(1), frame_spec(2),
                pl.BlockSpec((27, C, tn), lambda j, m: (0, 0, j)),
                pl.BlockSpec((1, tn), lambda j, m: (0, j))]
    operands = [x, x, x, w, shift.astype(jnp.float32).reshape(1, Cout)]
    if has_res:
        in_specs.append(pl.BlockSpec(
            (1, 1, Hp, Wp, tn), lambda j, m: (m // Tp, m % Tp, 0, 0, j)))
        operands.append(residual)

    body = functools.partial(_win_body, Tp=Tp, Ho=Ho, Wo=Wo, C=C, relu=relu,
                             has_res=has_res)
    flops = 2 * B * (Tp - 2) * Ho * Wo * 27 * C * Cout
    out = pl.pallas_call(
        body,
        out_shape=jax.ShapeDtypeStruct((B, Tp, Hp, Wp, Cout), jnp.bfloat16),
        grid=(nj, B * Tp),
        in_specs=in_specs,
        out_specs=pl.BlockSpec(
            (1, 1, Hp, Wp, tn), lambda j, m: (m // Tp, m % Tp, 0, 0, j)),
        compiler_params=pltpu.CompilerParams(
            dimension_semantics=("parallel", "arbitrary"),
            vmem_limit_bytes=_VMEM_LIMIT),
        cost_estimate=pl.CostEstimate(flops=flops, transcendentals=0,
                                      bytes_accessed=6 * B * Tp * Hp * Wp * C),
    )(*operands)
    return out


def _fold_weights(w_packed, c):
    cout = w_packed.shape[1]
    w3 = w_packed.reshape(3, 3, 3, c, cout)
    z = jnp.zeros((c, cout), w_packed.dtype)
    taps = []
    for kt in range(3):
        for a in range(2):
            for b in range(2):
                rows = []
                for ph in range(2):
                    for pw in range(2):
                        cols = []
                        for po in range(2):
                            for qo in range(2):
                                kh = 2 * a + ph - po
                                kw = 2 * b + pw - qo
                                ok = 0 <= kh < 3 and 0 <= kw < 3
                                cols.append(w3[kt, kh, kw] if ok else z)
                        rows.append(jnp.concatenate(cols, axis=1))
                taps.append(jnp.concatenate(rows, axis=0))
    return jnp.stack(taps)


def _fold_body(f0, f1, f2, w_ref, s_ref, m_ref, *rest, Tp, Q, relu, has_res):
    if has_res:
        res_ref, o_ref = rest
    else:
        (o_ref,) = rest
        res_ref = None
    tp = pl.program_id(0) % Tp
    Qc = Q - 2
    M = Qc * Q
    o_ref[...] = jnp.zeros_like(o_ref)

    @pl.when(jnp.logical_and(tp >= 1, tp <= Tp - 2))
    def _():
        frames = (f0, f1, f2)
        cf = w_ref.shape[1]
        acc = jnp.zeros((M, cf), jnp.float32)
        for kt in range(3):
            flat = frames[kt][0, 0].reshape(Q * Q, cf)
            for a in range(2):
                for b in range(2):
                    off = a * Q + b
                    acc = acc + jnp.dot(
                        flat[off:off + M],
                        w_ref[(kt * 2 + a) * 2 + b],
                        preferred_element_type=jnp.float32)
        y = acc + s_ref[...]
        if res_ref is not None:
            rflat = res_ref[0, 0].reshape(Q * Q, cf)
            y = y + rflat[Q + 1:Q + 1 + M].astype(jnp.float32)
        if relu:
            y = jnp.maximum(y, 0.0)
        y = y * m_ref[...]
        o_ref[0, 0, 0:Qc, :, :] = y.reshape(Qc, Q, cf).astype(o_ref.dtype)


def _conv_s1_folded(x, w_folded, shift4, mask, relu=True, residual=None):
    B, Tp, Q, _, CF = x.shape
    has_res = residual is not None

    def frame_spec(k):
        return pl.BlockSpec(
            (1, 1, Q, Q, CF),
            lambda m, k=k: (m // Tp,
                            jnp.clip(m % Tp - 1 + k, 0, Tp - 1), 0, 0, 0))

    in_specs = [frame_spec(0), frame_spec(1), frame_spec(2),
                pl.BlockSpec((12, CF, CF), lambda m: (0, 0, 0)),
                pl.BlockSpec((1, CF), lambda m: (0, 0)),
                pl.BlockSpec(((Q - 2) * Q, CF), lambda m: (0, 0))]
    operands = [x, x, x, w_folded, shift4.reshape(1, CF), mask]
    if has_res:
        in_specs.append(pl.BlockSpec(
            (1, 1, Q, Q, CF), lambda m: (m // Tp, m % Tp, 0, 0, 0)))
        operands.append(residual)

    body = functools.partial(_fold_body, Tp=Tp, Q=Q, relu=relu,
                             has_res=has_res)
    flops = 2 * B * (Tp - 2) * (Q - 1) * (Q - 1) * 12 * CF * CF
    out = pl.pallas_call(
        body,
        out_shape=jax.ShapeDtypeStruct((B, Tp, Q, Q, CF), jnp.bfloat16),
        grid=(B * Tp,),
        in_specs=in_specs,
        out_specs=pl.BlockSpec(
            (1, 1, Q, Q, CF), lambda m: (m // Tp, m % Tp, 0, 0, 0)),
        compiler_params=pltpu.CompilerParams(
            dimension_semantics=("arbitrary",),
            vmem_limit_bytes=_VMEM_LIMIT),
        cost_estimate=pl.CostEstimate(flops=flops, transcendentals=0,
                                      bytes_accessed=6 * B * Tp * Q * Q * CF),
    )(*operands)
    return out


def _fold_mask(Q, alpha):
    Qc = Q - 2
    H = 2 * Q - 8
    ph = jnp.arange(2)[None, :]
    oh = 2 * jnp.arange(Qc)[:, None] + ph - alpha
    ow = 2 * jnp.arange(Q)[:, None] + ph - alpha
    okh = jnp.logical_and(oh >= 0, oh <= H - 1)
    okw = jnp.logical_and(ow >= 0, ow <= H - 1)
    m = (okh[:, None, :, None] & okw[None, :, None, :]).astype(jnp.float32)
    return m.reshape(Qc * Q, 4)


def _fold_activation(x, c):
    B, T, H, W, _ = x.shape
    xp = jnp.pad(x, ((0, 0), (1, 1), (5, 3), (5, 3), (0, 0)))
    Q = (H + 8) // 2
    xp = xp.reshape(B, T + 2, Q, 2, Q, 2, c)
    xp = xp.transpose(0, 1, 2, 4, 3, 5, 6)
    return xp.reshape(B, T + 2, Q, Q, 4 * c)


def _unfold_activation(xf, c, alpha):
    B, Tp, Q, _, _ = xf.shape
    H = 2 * Q - 8
    x = xf.reshape(B, Tp, Q, Q, 2, 2, c)
    x = x.transpose(0, 1, 2, 4, 3, 5, 6)
    x = x.reshape(B, Tp, 2 * Q, 2 * Q, c)
    return x[:, :, alpha - 1:alpha + H + 1, alpha - 1:alpha + H + 1, :]


def _im2col_s2(xp, To, Ho, Wo, kt_n=3, kh_n=3, kw_n=3):
    C = xp.shape[-1]
    B = xp.shape[0]
    cols = []
    for kt in range(kt_n):
        for kh in range(kh_n):
            for kw in range(kw_n):
                cols.append(xp[:, kt:kt + 2 * To - 1:2,
                               kh:kh + 2 * Ho - 1:2,
                               kw:kw + 2 * Wo - 1:2, :])
    P = jnp.stack(cols, axis=4)
    return P.reshape(B * To * Ho * Wo, kt_n * kh_n * kw_n * C)


def _pad_act(y, B, To, Ho, Wo, C):
    y = y.reshape(B, To, Ho, Wo, C)
    return jnp.pad(y, ((0, 0), (1, 1), (1, 2), (1, 1), (0, 0)))


def _block_pair(xpad, w1, s1, w2, s2, down=None):
    out = _conv_s1(xpad, w1, s1, relu=True)
    identity = xpad if down is None else down
    return _conv_s1(out, w2, s2, relu=True, residual=identity)


def kernel(x, stem_w, stem_shift,
           layer1_b0_conv1_w, layer1_b0_conv1_shift,
           layer1_b0_conv2_w, layer1_b0_conv2_shift,
           layer1_b1_conv1_w, layer1_b1_conv1_shift,
           layer1_b1_conv2_w, layer1_b1_conv2_shift,
           layer2_b0_conv1_w, layer2_b0_conv1_shift,
           layer2_b0_conv2_w, layer2_b0_conv2_shift,
           layer2_b0_down_w, layer2_b0_down_shift,
           layer2_b1_conv1_w, layer2_b1_conv1_shift,
           layer2_b1_conv2_w, layer2_b1_conv2_shift,
           layer3_b0_conv1_w, layer3_b0_conv1_shift,
           layer3_b0_conv2_w, layer3_b0_conv2_shift,
           layer3_b0_down_w, layer3_b0_down_shift,
           layer3_b1_conv1_w, layer3_b1_conv1_shift,
           layer3_b1_conv2_w, layer3_b1_conv2_shift,
           layer4_b0_conv1_w, layer4_b0_conv1_shift,
           layer4_b0_conv2_w, layer4_b0_conv2_shift,
           layer4_b0_down_w, layer4_b0_down_shift,
           layer4_b1_conv1_w, layer4_b1_conv1_shift,
           layer4_b1_conv2_w, layer4_b1_conv2_shift,
           proj_w, proj_shift):
    B = x.shape[0]

    xs = jnp.transpose(x, (0, 2, 3, 4, 1)).astype(jnp.bfloat16)[..., 0]
    xs = jnp.pad(xs, ((0, 0), (1, 1), (3, 3), (3, 3)))
    cols = []
    for kt in range(3):
        for kh in range(7):
            for kw in range(7):
                cols.append(xs[:, kt:kt + 16,
                               kh:kh + 111:2, kw:kw + 111:2])
    A = jnp.stack(cols, axis=-1).reshape(B * 16 * 56 * 56, 147)
    stem = _matmul(A, stem_w, stem_shift, relu=True)

    f = _fold_activation(stem.reshape(B, 16, 56, 56, 64), 64)
    Q = f.shape[2]
    masks = [jnp.repeat(_fold_mask(Q, a), 64, axis=1) for a in (4, 3, 2, 1)]
    sh = lambda s: jnp.tile(s.astype(jnp.float32), 4)
    w10c1 = _fold_weights(layer1_b0_conv1_w, 64)
    w10c2 = _fold_weights(layer1_b0_conv2_w, 64)
    w11c1 = _fold_weights(layer1_b1_conv1_w, 64)
    w11c2 = _fold_weights(layer1_b1_conv2_w, 64)
    t = _conv_s1_folded(f, w10c1, sh(layer1_b0_conv1_shift), masks[0])
    f1o = _conv_s1_folded(t, w10c2, sh(layer1_b0_conv2_shift), masks[1],
                          residual=f)
    t = _conv_s1_folded(f1o, w11c1, sh(layer1_b1_conv1_shift), masks[2])
    act1 = _conv_s1_folded(t, w11c2, sh(layer1_b1_conv2_shift), masks[3],
                           residual=f1o)
    act1u = _unfold_activation(act1, 64, 1)

    A = _im2col_s2(act1u, 8, 28, 28)
    y = _matmul(A, layer2_b0_conv1_w, layer2_b0_conv1_shift, relu=True)
    c1p = _pad_act(y, B, 8, 28, 28, 128)
    Ad = act1u[:, 1:16:2, 1:56:2, 1:56:2, :].reshape(-1, 64)
    yd = _matmul(Ad, layer2_b0_down_w, layer2_b0_down_shift, relu=False)
    dpad = _pad_act(yd, B, 8, 28, 28, 128)
    b0 = _conv_s1(c1p, layer2_b0_conv2_w, layer2_b0_conv2_shift, relu=True,
                  residual=dpad)
    act2 = _block_pair(b0, layer2_b1_conv1_w, layer2_b1_conv1_shift,
                       layer2_b1_conv2_w, layer2_b1_conv2_shift)

    A = _im2col_s2(act2, 4, 14, 14)
    y = _matmul(A, layer3_b0_conv1_w, layer3_b0_conv1_shift, relu=True)
    c1p = _pad_act(y, B, 4, 14, 14, 256)
    Ad = act2[:, 1:8:2, 1:28:2, 1:28:2, :].reshape(-1, 128)
    yd = _matmul(Ad, layer3_b0_down_w, layer3_b0_down_shift, relu=False)
    dpad = _pad_act(yd, B, 4, 14, 14, 256)
    b0 = _conv_s1(c1p, layer3_b0_conv2_w, layer3_b0_conv2_shift, relu=True,
                  residual=dpad)
    act3 = _block_pair(b0, layer3_b1_conv1_w, layer3_b1_conv1_shift,
                       layer3_b1_conv2_w, layer3_b1_conv2_shift)

    A = _im2col_s2(act3, 2, 7, 7)
    y = _matmul(A, layer4_b0_conv1_w, layer4_b0_conv1_shift, relu=True)
    c1p = _pad_act(y, B, 2, 7, 7, 512)
    Ad = act3[:, 1:4:2, 1:14:2, 1:14:2, :].reshape(-1, 256)
    yd = _matmul(Ad, layer4_b0_down_w, layer4_b0_down_shift, relu=False)
    dpad = _pad_act(yd, B, 2, 7, 7, 512)
    b0 = _conv_s1(c1p, layer4_b0_conv2_w, layer4_b0_conv2_shift, relu=True,
                  residual=dpad)
    act4 = _block_pair(b0, layer4_b1_conv1_w, layer4_b1_conv1_shift,
                       layer4_b1_conv2_w, layer4_b1_conv2_shift)

    feats = jnp.mean(act4[:, 1:3, 1:8, 1:8, :].astype(jnp.float32),
                     axis=(1, 2, 3))
    emb = _matmul(feats, proj_w, proj_shift, relu=False, out_dtype=jnp.float32)
    return emb
